# baseline (speedup 1.0000x reference)
import dataclasses
import functools

import jax
import jax.numpy as jnp
from jax import lax
from jax.experimental import pallas as pl
from jax.experimental.pallas import tpu as pltpu
from jax.experimental.pallas import tpu_sc as plsc

_NX, _NY, _NZ, _C = 432, 496, 1, 64
_CW = 128
_P = 12000
_PPAD = 12288
_B = 4
_K = 4096
_G = _NX * _NY
_CBLK = 3968
_NBLK = 56
_GP = _CBLK * _NBLK
_NW = 32
_WIN = 128
_WPW = (_B * _PPAD) // (_NW * _WIN)
_SPW = 2 * _GP
_ZCH = _SPW // 16


def _sc_compiler_params():
    cp = pltpu.CompilerParams()
    if "needs_layout_passes" in pltpu.CompilerParams.__dataclass_fields__:
        cp = dataclasses.replace(cp, needs_layout_passes=False)
    return cp


def _compute_body(pf_ref, co_ref, w1_ref, b1_ref, g1_ref, be1_ref, m1_ref,
                  v1_ref, ws_ref, sc_ref, acts_ref, dg_ref, s3_ref):
    b = pl.program_id(0)
    x = pf_ref[...]
    h = lax.dot_general(x, w1_ref[...], (((1,), (1,)), ((), ())),
                        preferred_element_type=jnp.float32) + b1_ref[...]
    h = (h - m1_ref[...]) / jnp.sqrt(v1_ref[...] + 1e-5) * g1_ref[...] \
        + be1_ref[...]
    a = jax.nn.sigmoid(h)
    acts_ref[0:_P, 0:_C] = a
    acts_ref[0:_P, _C:_CW] = jnp.zeros((_P, _CW - _C), jnp.float32)
    acts_ref[_P:_PPAD, :] = jnp.zeros((_PPAD - _P, _CW), jnp.float32)

    s0 = lax.dot_general(ws_ref[...], a, (((1,), (1,)), ((), ())),
                         preferred_element_type=jnp.float32) + sc_ref[0]
    s0 = (s0 - sc_ref[3]) / jnp.sqrt(sc_ref[4] + 1e-5) * sc_ref[1] + sc_ref[2]
    score = jax.nn.sigmoid(s0)

    bits = lax.bitcast_convert_type(score, jnp.int32)

    def bs_body(_, lh):
        lo, hi = lh
        mid = lo + 1 + ((hi - lo - 1) >> 1)
        cnt = jnp.sum((bits >= mid).astype(jnp.int32))
        ok = cnt >= _K
        return (jnp.where(ok, mid, lo), jnp.where(ok, hi, mid - 1))

    tau, _ = lax.fori_loop(0, 31, bs_body,
                           (jnp.int32(0), jnp.int32(2147483647)))

    n_gt = jnp.sum((bits > tau).astype(jnp.int32))
    need = _K - n_gt
    is_tie = bits == tau
    iota = lax.broadcasted_iota(jnp.int32, (1, _P), 1)

    def ts_body(_, lh):
        lo2, hi2 = lh
        mid = (lo2 + hi2) >> 1
        cnt = jnp.sum((is_tie & (iota < mid)).astype(jnp.int32))
        ok = cnt >= need
        return (jnp.where(ok, lo2, mid + 1), jnp.where(ok, mid, hi2))

    m, _ = lax.fori_loop(0, 14, ts_body, (jnp.int32(1), jnp.int32(_P)))
    sel = (bits > tau) | (is_tie & (iota < m))

    cells = (co_ref[0, 1:2, 0:_P] + co_ref[0, 2:3, 0:_P] * _NX
             + co_ref[0, 3:4, 0:_P])
    dg_ref[0, 0:1, 0:_P] = jnp.where(sel, b * _GP + cells, b * _GP + _G)
    dg_ref[0, 0:1, _P:_PPAD] = jnp.full((1, _PPAD - _P), _G, jnp.int32) \
        + b * _GP

    tot = jnp.sum(score)
    ssel = jnp.sum(jnp.where(sel, score, 0.0))
    s3_ref[...] = jnp.full((1, 1, 128), (tot - ssel) / float(_P - _K),
                           jnp.float32)


def _expand_body(outt_ref, stamp_ref, o_ref):
    j = pl.program_id(1)
    xt = outt_ref[:, 0:_C].T
    srows = stamp_ref[0, pl.ds(j * (_CBLK // 128), _CBLK // 128), :]
    xt3 = xt.reshape(_C, _CBLK // 128, 128)
    o = jnp.where(srows[None] != 0, xt3, 0.0)
    o_ref[...] = o.reshape(1, _C, _CBLK)


def kernel(pillar_features, voxel_coords, W1, b1, g1, be1, m1, v1,
           Ws, bs, gs, bes, ms, vs):
    f32, i32 = jnp.float32, jnp.int32
    co_t = jnp.pad(
        voxel_coords.astype(i32).reshape(_B, _P, 4).transpose(0, 2, 1),
        ((0, 0), (0, 0), (0, _PPAD - _P)))
    row = lambda v: v.reshape(1, -1).astype(f32)
    scal = jnp.concatenate([bs.reshape(1), gs.reshape(1), bes.reshape(1),
                            ms.reshape(1), vs.reshape(1)]).astype(f32)

    _call_compute = pl.pallas_call(
        _compute_body,
        grid=(_B,),
        in_specs=[
            pl.BlockSpec((_P, _C), lambda b: (b, 0)),
            pl.BlockSpec((1, 4, _PPAD), lambda b: (b, 0, 0)),
            pl.BlockSpec((_C, _C), lambda b: (0, 0)),
        ] + [pl.BlockSpec((1, _C), lambda b: (0, 0))] * 5
          + [pl.BlockSpec((1, _C), lambda b: (0, 0))]
          + [pl.BlockSpec(memory_space=pltpu.SMEM)],
        out_specs=[
            pl.BlockSpec((_PPAD, _CW), lambda b: (b, 0)),
            pl.BlockSpec((1, 1, _PPAD), lambda b: (b, 0, 0)),
            pl.BlockSpec((1, 1, 128), lambda b: (b, 0, 0)),
        ],
        out_shape=[
            jax.ShapeDtypeStruct((_B * _PPAD, _CW), f32),
            jax.ShapeDtypeStruct((_B, 1, _PPAD), i32),
            jax.ShapeDtypeStruct((_B, 1, 128), f32),
        ],
    )
    acts, dg, s3 = _call_compute(
        pillar_features, co_t, W1.astype(f32), row(b1), row(g1), row(be1),
        row(m1), row(v1), Ws.reshape(1, _C).astype(f32), scal)

    acts3 = acts.reshape(_NW * _WPW, _WIN, _CW)
    dg3 = dg.reshape(_NW * _WPW, 1, _WIN)
    dgc = dg.reshape(2, 2 * _PPAD)
    zeros_z = jnp.zeros((_ZCH,), i32)

    mesh = plsc.VectorSubcoreMesh(core_axis_name="c", subcore_axis_name="s")

    @functools.partial(
        pl.kernel,
        out_type=jax.ShapeDtypeStruct((_B * _GP, _CW), f32),
        mesh=mesh,
        scratch_types=[pltpu.VMEM((1, _WIN), i32),
                       pltpu.VMEM((_WIN, _CW), f32)],
    )
    def _scatter_kernel(acts_hbm, dg_hbm, outt_hbm, idxg_v, rows_v):
        c = lax.axis_index("c")
        s = lax.axis_index("s")
        w = c * 16 + s

        @pl.loop(0, _WPW)
        def _(j):
            t = w * _WPW + j
            pltpu.sync_copy(dg_hbm.at[t], idxg_v)
            pltpu.sync_copy(acts_hbm.at[t], rows_v)
            pltpu.sync_copy(rows_v, outt_hbm.at[idxg_v.at[0]])

    @functools.partial(
        pl.kernel,
        out_type=jax.ShapeDtypeStruct((_B * _GP,), i32),
        mesh=mesh,
        scratch_types=[pltpu.VMEM((2 * _PPAD,), i32),
                       pltpu.VMEM((_ZCH,), i32)],
        compiler_params=_sc_compiler_params(),
    )
    def _stamp_kernel(dgc_hbm, zeros_hbm, stamp_hbm, dgc_v, stamp_v):
        c = lax.axis_index("c")
        s = lax.axis_index("s")
        gbase = c * _SPW + s * _ZCH

        pltpu.sync_copy(zeros_hbm, stamp_v)
        pltpu.sync_copy(dgc_hbm.at[c], dgc_v)
        ones16 = jnp.full((16,), 1, jnp.int32)

        @pl.loop(0, 2 * _PPAD, step=16)
        def _(i):
            dlv = dgc_v[pl.ds(i, 16)]
            mask = (dlv >= gbase) & (dlv < gbase + _ZCH)
            il = jnp.minimum(jnp.maximum(dlv - gbase, 0), _ZCH - 1)
            plsc.store_scatter(stamp_v, [il], ones16, mask=mask)

        pltpu.sync_copy(stamp_v, stamp_hbm.at[pl.ds(gbase, _ZCH)])

    out_t = _scatter_kernel(acts3, dg3)
    stamp = _stamp_kernel(dgc, zeros_z)

    out3 = pl.pallas_call(
        _expand_body,
        grid=(_B, _G // _CBLK),
        in_specs=[
            pl.BlockSpec((_CBLK, _CW), lambda b, j: (b * _NBLK + j, 0)),
            pl.BlockSpec((1, _GP // 128, 128), lambda b, j: (b, 0, 0)),
        ],
        out_specs=pl.BlockSpec((1, _C, _CBLK), lambda b, j: (b, 0, j)),
        out_shape=jax.ShapeDtypeStruct((_B, _C, _G), f32),
    )(out_t, stamp.reshape(_B, _GP // 128, 128))

    return out3.reshape(_B, _C * _NZ, _NY, _NX), s3[:, 0, 0]


# speedup vs baseline: 1.8370x; 1.8370x over previous
import dataclasses
import functools

import jax
import jax.numpy as jnp
from jax import lax
from jax.experimental import pallas as pl
from jax.experimental.pallas import tpu as pltpu
from jax.experimental.pallas import tpu_sc as plsc

_NX, _NY, _NZ, _C = 432, 496, 1, 64
_CW = 128
_P = 12000
_PPAD = 12288
_B = 4
_K = 4096
_G = _NX * _NY
_CBLK = 3968
_NBLK = 56
_GP = _CBLK * _NBLK
_NW = 32
_WIN = 128
_WPW = (_B * _PPAD) // (_NW * _WIN)
_SPW = 2 * _GP
_ZCH = _SPW // 16


def _sc_compiler_params():
    cp = pltpu.CompilerParams()
    if "needs_layout_passes" in pltpu.CompilerParams.__dataclass_fields__:
        cp = dataclasses.replace(cp, needs_layout_passes=False)
    return cp


def _compute_body(pf_ref, co_ref, w1_ref, b1_ref, g1_ref, be1_ref, m1_ref,
                  v1_ref, ws_ref, sc_ref, acts_ref, dg_ref, s3_ref):
    b = pl.program_id(0)
    x = pf_ref[...]
    h = lax.dot_general(x, w1_ref[...], (((1,), (1,)), ((), ())),
                        preferred_element_type=jnp.float32) + b1_ref[...]
    h = (h - m1_ref[...]) / jnp.sqrt(v1_ref[...] + 1e-5) * g1_ref[...] \
        + be1_ref[...]
    a = jax.nn.sigmoid(h)
    acts_ref[0:_P, 0:_C] = a
    acts_ref[0:_P, _C:_CW] = jnp.zeros((_P, _CW - _C), jnp.float32)
    acts_ref[_P:_PPAD, :] = jnp.zeros((_PPAD - _P, _CW), jnp.float32)

    s0 = lax.dot_general(ws_ref[...], a, (((1,), (1,)), ((), ())),
                         preferred_element_type=jnp.float32) + sc_ref[0]
    s0 = (s0 - sc_ref[3]) / jnp.sqrt(sc_ref[4] + 1e-5) * sc_ref[1] + sc_ref[2]
    score = jax.nn.sigmoid(s0)

    bits = lax.bitcast_convert_type(score, jnp.int32)

    def bs_body(_, lh):
        lo, hi = lh
        mid = lo + 1 + ((hi - lo - 1) >> 1)
        cnt = jnp.sum((bits >= mid).astype(jnp.int32))
        ok = cnt >= _K
        return (jnp.where(ok, mid, lo), jnp.where(ok, hi, mid - 1))

    tau, _ = lax.fori_loop(0, 31, bs_body,
                           (jnp.int32(0), jnp.int32(2147483647)))

    n_gt = jnp.sum((bits > tau).astype(jnp.int32))
    need = _K - n_gt
    is_tie = bits == tau
    iota = lax.broadcasted_iota(jnp.int32, (1, _P), 1)

    def ts_body(_, lh):
        lo2, hi2 = lh
        mid = (lo2 + hi2) >> 1
        cnt = jnp.sum((is_tie & (iota < mid)).astype(jnp.int32))
        ok = cnt >= need
        return (jnp.where(ok, lo2, mid + 1), jnp.where(ok, mid, hi2))

    m, _ = lax.fori_loop(0, 14, ts_body, (jnp.int32(1), jnp.int32(_P)))
    sel = (bits > tau) | (is_tie & (iota < m))

    cells = (co_ref[0, 1:2, 0:_P] + co_ref[0, 3:4, 0:_P] * _NY
             + co_ref[0, 2:3, 0:_P])
    dg_ref[0, 0:1, 0:_P] = jnp.where(sel, b * _GP + cells, b * _GP + _G)
    dg_ref[0, 0:1, _P:_PPAD] = jnp.full((1, _PPAD - _P), _G, jnp.int32) \
        + b * _GP

    tot = jnp.sum(score)
    ssel = jnp.sum(jnp.where(sel, score, 0.0))
    s3_ref[...] = jnp.full((1, 1, 128), (tot - ssel) / float(_P - _K),
                           jnp.float32)


def _expand_body(outt_ref, stamp_ref, o_ref):
    j = pl.program_id(1)
    xt = outt_ref[:, 0:_C].T
    srows = stamp_ref[0, pl.ds(j * (_CBLK // 128), _CBLK // 128), :]
    xt3 = xt.reshape(_C, _CBLK // 128, 128)
    o = jnp.where(srows[None] != 0, xt3, 0.0)
    o_ref[...] = o.reshape(1, _C, _CBLK)


def kernel(pillar_features, voxel_coords, W1, b1, g1, be1, m1, v1,
           Ws, bs, gs, bes, ms, vs):
    f32, i32 = jnp.float32, jnp.int32
    co_t = jnp.pad(
        voxel_coords.astype(i32).reshape(_B, _P, 4).transpose(0, 2, 1),
        ((0, 0), (0, 0), (0, _PPAD - _P)))
    row = lambda v: v.reshape(1, -1).astype(f32)
    scal = jnp.concatenate([bs.reshape(1), gs.reshape(1), bes.reshape(1),
                            ms.reshape(1), vs.reshape(1)]).astype(f32)

    _call_compute = pl.pallas_call(
        _compute_body,
        grid=(_B,),
        in_specs=[
            pl.BlockSpec((_P, _C), lambda b: (b, 0)),
            pl.BlockSpec((1, 4, _PPAD), lambda b: (b, 0, 0)),
            pl.BlockSpec((_C, _C), lambda b: (0, 0)),
        ] + [pl.BlockSpec((1, _C), lambda b: (0, 0))] * 5
          + [pl.BlockSpec((1, _C), lambda b: (0, 0))]
          + [pl.BlockSpec(memory_space=pltpu.SMEM)],
        out_specs=[
            pl.BlockSpec((_PPAD, _CW), lambda b: (b, 0)),
            pl.BlockSpec((1, 1, _PPAD), lambda b: (b, 0, 0)),
            pl.BlockSpec((1, 1, 128), lambda b: (b, 0, 0)),
        ],
        out_shape=[
            jax.ShapeDtypeStruct((_B * _PPAD, _CW), f32),
            jax.ShapeDtypeStruct((_B, 1, _PPAD), i32),
            jax.ShapeDtypeStruct((_B, 1, 128), f32),
        ],
    )
    acts, dg, s3 = _call_compute(
        pillar_features, co_t, W1.astype(f32), row(b1), row(g1), row(be1),
        row(m1), row(v1), Ws.reshape(1, _C).astype(f32), scal)

    acts3 = acts.reshape(_NW * _WPW, _WIN, _CW)
    dg3 = dg.reshape(_NW * _WPW, 1, _WIN)
    dgc = dg.reshape(2, 2 * _PPAD)
    zeros_z = jnp.zeros((_ZCH,), i32)

    mesh = plsc.VectorSubcoreMesh(core_axis_name="c", subcore_axis_name="s")

    @functools.partial(
        pl.kernel,
        out_type=jax.ShapeDtypeStruct((_B * _GP, _CW), f32),
        mesh=mesh,
        scratch_types=[pltpu.VMEM((1, _WIN), i32),
                       pltpu.VMEM((_WIN, _CW), f32)],
    )
    def _scatter_kernel(acts_hbm, dg_hbm, outt_hbm, idxg_v, rows_v):
        c = lax.axis_index("c")
        s = lax.axis_index("s")
        w = c * 16 + s

        @pl.loop(0, _WPW)
        def _(j):
            t = w * _WPW + j
            pltpu.sync_copy(dg_hbm.at[t], idxg_v)
            pltpu.sync_copy(acts_hbm.at[t], rows_v)
            pltpu.sync_copy(rows_v, outt_hbm.at[idxg_v.at[0]])

    @functools.partial(
        pl.kernel,
        out_type=jax.ShapeDtypeStruct((_B * _GP,), i32),
        mesh=mesh,
        scratch_types=[pltpu.VMEM((2 * _PPAD,), i32),
                       pltpu.VMEM((_ZCH,), i32)],
        compiler_params=_sc_compiler_params(),
    )
    def _stamp_kernel(dgc_hbm, zeros_hbm, stamp_hbm, dgc_v, stamp_v):
        c = lax.axis_index("c")
        s = lax.axis_index("s")
        gbase = c * _SPW + s * _ZCH

        pltpu.sync_copy(zeros_hbm, stamp_v)
        pltpu.sync_copy(dgc_hbm.at[c], dgc_v)
        ones16 = jnp.full((16,), 1, jnp.int32)

        @pl.loop(0, 2 * _PPAD, step=16)
        def _(i):
            dlv = dgc_v[pl.ds(i, 16)]
            mask = (dlv >= gbase) & (dlv < gbase + _ZCH)
            il = jnp.minimum(jnp.maximum(dlv - gbase, 0), _ZCH - 1)
            plsc.store_scatter(stamp_v, [il], ones16, mask=mask)

        pltpu.sync_copy(stamp_v, stamp_hbm.at[pl.ds(gbase, _ZCH)])

    out_t = _scatter_kernel(acts3, dg3)
    stamp = _stamp_kernel(dgc, zeros_z)

    out3 = pl.pallas_call(
        _expand_body,
        grid=(_B, _G // _CBLK),
        in_specs=[
            pl.BlockSpec((_CBLK, _CW), lambda b, j: (b * _NBLK + j, 0)),
            pl.BlockSpec((1, _GP // 128, 128), lambda b, j: (b, 0, 0)),
        ],
        out_specs=pl.BlockSpec((1, _C, _CBLK), lambda b, j: (b, 0, j)),
        out_shape=jax.ShapeDtypeStruct((_B, _C, _G), f32),
    )(out_t, stamp.reshape(_B, _GP // 128, 128))

    out4 = jnp.swapaxes(out3.reshape(_B, _C * _NZ, _NX, _NY), 2, 3)
    return out4, s3[:, 0, 0]
